# baseline (speedup 1.0000x reference)
import functools

import jax
import jax.numpy as jnp
from jax import lax
from jax.experimental import pallas as pl
from jax.experimental.pallas import tpu as pltpu
from jax.experimental.pallas import tpu_sc as plsc

_W = 4096
_KW = 16
_CH = 64


def _emb_lookup_cm(idx, tab3, tail3):
    A, R, V = tab3.shape
    D = A * R
    B, = idx.shape
    info = plsc.get_sparse_core_info()
    NC, NS = info.num_cores, info.num_subcores
    a_per_c = A // NC
    n_grp = B // 16
    cap = B + 16
    cap2 = B + 256
    P = B + NS * _KW * _CH
    n_ph = a_per_c * _KW
    last_w0 = V // _W * _W
    last_len = (V - last_w0) // 128 * 128
    v_max = last_w0 + last_len
    mesh = plsc.VectorSubcoreMesh(core_axis_name="c", subcore_axis_name="s")

    @functools.partial(
        pl.kernel,
        mesh=mesh,
        out_type=(jax.ShapeDtypeStruct((D * P,), jnp.float32),
                  jax.ShapeDtypeStruct((P,), jnp.int32)),
        scratch_types=[
            pltpu.VMEM((B,), jnp.int32),
            pltpu.VMEM((cap,), jnp.int32),
            pltpu.VMEM((cap2,), jnp.int32),
            pltpu.VMEM((16,), jnp.int32),
            pltpu.VMEM((16,), jnp.int32),
            pltpu.VMEM((2, R, _W), jnp.float32),
            pltpu.VMEM((16,), jnp.int32),
            pltpu.SMEM((8,), jnp.int32),
        ] + [pltpu.VMEM((_CH * 8,), jnp.float32) for _ in range(2)] + [
            pltpu.VMEM((_CH,), jnp.int32) for _ in range(2)
        ] + [
            pltpu.SemaphoreType.DMA,
            pltpu.SemaphoreType.DMA,
        ],
        compiler_params=pltpu.CompilerParams(needs_layout_passes=False),
    )
    def emb(idx_hbm, tab_hbm, tail_hbm, vals_hbm, pos_hbm, idx_all, lst, lst2,
            off_v, cnt_v, seg_v, off2_v, smem, *rest):
        vb = [rest[0], rest[1]]
        jb = [rest[2], rest[3]]
        ssem, gsem = rest[4], rest[5]
        c = lax.axis_index("c")
        s = lax.axis_index("s")
        lanes = lax.iota(jnp.int32, 16)

        pltpu.sync_copy(idx_hbm, idx_all)

        def init(g, carry):
            lst[pl.ds(g * 16, 16)] = jnp.full((16,), B << 16, jnp.int32)
            return carry

        lax.fori_loop(0, cap // 16, init, 0)

        def scan(g, n):
            v = idx_all[pl.ds(g * 16, 16)]
            tl = v >= v_max
            m = jnp.where(tl, s == NS - 1,
                          lax.shift_right_logical(v, 16) == s)
            kf = jnp.where(tl, _KW - 1,
                           lax.shift_right_logical(v, 12) & (_KW - 1))
            loc = jnp.where(tl, v - v_max, v & (_W - 1))
            pk = lax.shift_left(lanes + g * 16, 16) | loc | (
                lax.shift_left(kf, 12))
            plsc.store_compressed(lst.at[pl.ds(n, 16)], pk, mask=m)
            return n + plsc.all_reduce_population_count(m)[0]

        n = lax.fori_loop(0, n_grp, scan, 0)
        ng = lax.shift_right_logical(n + 15, 4)

        offs = jnp.zeros((16,), jnp.int32)
        cnts = jnp.zeros((16,), jnp.int32)
        cur = 0
        for k in range(_KW):
            def split(g, m_cur, k=k):
                pk = lst[pl.ds(g * 16, 16)]
                m = (lax.shift_right_logical(pk, 12) & (_KW - 1)) == k
                plsc.store_compressed(lst2.at[pl.ds(m_cur, 16)], pk, mask=m)
                return m_cur + plsc.all_reduce_population_count(m)[0]

            nxt = lax.fori_loop(0, ng, split, cur)
            offs = jnp.where(lanes == k, cur, offs)
            cnts = jnp.where(lanes == k, nxt - cur, cnts)
            cur = nxt
        off_v[...] = offs
        cnt_v[...] = cnts

        offs2 = jnp.zeros((16,), jnp.int32)
        cur2 = 0
        for k in range(_KW):
            offs2 = jnp.where(lanes == k, cur2, offs2)
            nk2 = plsc.load_gather(cnt_v,
                                   [jnp.full((16,), k, jnp.int32)])[0]
            cur2 = cur2 + lax.shift_left(
                lax.shift_right_logical(nk2 + _CH - 1, 6), 6)
        off2_v[...] = offs2
        smem[0] = 0
        plsc.subcore_barrier()
        for t in range(NS):
            @pl.when(s < t)
            def _(t=t):
                plsc.fetch_and_add(smem.at[0], cur2, subcore_id=t)
        plsc.subcore_barrier()
        pref = smem[0]

        def stage_parts(p):
            k = p & (_KW - 1)
            kv = jnp.broadcast_to(k, (16,)).astype(jnp.int32)
            n_k = plsc.load_gather(cnt_v, [kv])[0]
            w0 = pl.multiple_of((s * _KW + k) * _W, 128)
            a = c * a_per_c + lax.shift_right_logical(p, 4)
            return n_k, w0, a

        def stage_issue(p, slot):
            n_k, w0, a = stage_parts(p)
            k = p & (_KW - 1)
            is_tail = (s == NS - 1) & (k == _KW - 1)

            @pl.when((n_k > 0) & (w0 < last_w0) & (~is_tail))
            def _():
                pltpu.async_copy(tab_hbm.at[a, :, pl.ds(w0, _W)],
                                 seg_v.at[slot], gsem)

            @pl.when((n_k > 0) & (w0 == last_w0))
            def _():
                pltpu.async_copy(
                    tab_hbm.at[a, :, pl.ds(last_w0, last_len)],
                    seg_v.at[slot, :, pl.ds(0, last_len)], gsem)

            @pl.when((n_k > 0) & is_tail)
            def _():
                pltpu.async_copy(tail_hbm.at[a],
                                 seg_v.at[slot, :, pl.ds(0, 128)], gsem)

        def stage_wait(p, slot):
            n_k, w0, a = stage_parts(p)
            k = p & (_KW - 1)
            is_tail = (s == NS - 1) & (k == _KW - 1)

            @pl.when((n_k > 0) & (w0 < last_w0) & (~is_tail))
            def _():
                pltpu.make_async_copy(tab_hbm.at[a, :, pl.ds(w0, _W)],
                                      seg_v.at[slot], gsem).wait()

            @pl.when((n_k > 0) & (w0 == last_w0))
            def _():
                pltpu.make_async_copy(
                    tab_hbm.at[a, :, pl.ds(last_w0, last_len)],
                    seg_v.at[slot, :, pl.ds(0, last_len)], gsem).wait()

            @pl.when((n_k > 0) & is_tail)
            def _():
                pltpu.make_async_copy(
                    tail_hbm.at[a],
                    seg_v.at[slot, :, pl.ds(0, 128)], gsem).wait()

        stage_issue(0, 0)

        def phase(p, pcarry):
            sg = p & 1
            stage_wait(p, sg)

            @pl.when(p + 1 < n_ph)
            def _():
                stage_issue(p + 1, 1 - sg)

            a = c * a_per_c + lax.shift_right_logical(p, 4)
            k = p & (_KW - 1)
            kvec = jnp.broadcast_to(k, (16,)).astype(jnp.int32)
            n_k = plsc.load_gather(cnt_v, [kvec])[0]
            o_k = plsc.load_gather(off_v, [kvec])[0]
            base = pl.multiple_of(
                pref + plsc.load_gather(off2_v, [kvec])[0], 64)
            nch = lax.shift_right_logical(n_k + _CH - 1, 6)
            vbase0 = a * (P * R)
            sgv = jnp.broadcast_to(sg, (16,)).astype(jnp.int32)

            def chunk(ch, carry):
                wbase = pl.multiple_of(base + ch * _CH, 64)
                vdst = pl.multiple_of(vbase0 + wbase * R, 64)
                for slot in range(2):
                    @pl.when((ch & 1) == slot)
                    def _(slot=slot):
                        @pl.when(ch >= 2)
                        def _():
                            pltpu.make_async_copy(
                                vb[slot],
                                vals_hbm.at[pl.ds(vdst, _CH * 8)],
                                ssem).wait()
                            pltpu.make_async_copy(
                                jb[slot], pos_hbm.at[pl.ds(wbase, _CH)],
                                ssem).wait()

                        for g2 in range(_CH // 16):
                            e0 = ch * _CH + g2 * 16
                            pk = lst2[pl.ds(o_k + e0, 16)]
                            j = lax.shift_right_logical(pk, 16)
                            loc = pk & (_W - 1)
                            ok = ((lanes + e0) < n_k) & (j < B)
                            tgt = (lanes + g2 * 16) * 8
                            for dd in range(R):
                                vals = plsc.load_gather(
                                    seg_v,
                                    [sgv, jnp.full((16,), dd, jnp.int32),
                                     loc])
                                plsc.store_scatter(vb[slot], [tgt + dd],
                                                   vals)
                            jb[slot][pl.ds(g2 * 16, 16)] = jnp.where(
                                ok, j, B)

                        pltpu.async_copy(
                            vb[slot],
                            vals_hbm.at[pl.ds(vdst, _CH * 8)], ssem)
                        pltpu.async_copy(
                            jb[slot], pos_hbm.at[pl.ds(wbase, _CH)], ssem)
                return carry

            lax.fori_loop(0, nch, chunk, 0)

            def drain(ch, dcarry):
                wbase = pl.multiple_of(base + ch * _CH, 64)
                vdst = pl.multiple_of(vbase0 + wbase * R, 64)
                for slot in range(2):
                    @pl.when((ch & 1) == slot)
                    def _(slot=slot):
                        pltpu.make_async_copy(
                            vb[slot],
                            vals_hbm.at[pl.ds(vdst, _CH * 8)],
                            ssem).wait()
                        pltpu.make_async_copy(
                            jb[slot], pos_hbm.at[pl.ds(wbase, _CH)],
                            ssem).wait()
                return dcarry

            lax.fori_loop(jnp.maximum(nch - 2, 0), nch, drain, 0)
            return pcarry

        lax.fori_loop(0, n_ph, phase, 0)

    return emb(idx, tab3, tail3)


def kernel(input, table):
    idx = input.astype(jnp.int32)
    V, D = table.shape
    B, = idx.shape
    v_max = V // 4096 * 4096 + (V - V // 4096 * 4096) // 128 * 128
    tail3 = jnp.pad(table[v_max:].T,
                    ((0, 0), (0, 128 - (V - v_max)))).reshape(8, 8, 128)
    vals1, pos = _emb_lookup_cm(idx, table.T.reshape(8, 8, V), tail3)
    P = pos.shape[0]
    vals = vals1.reshape(8, P, 8).transpose(1, 0, 2).reshape(P, D)
    ar = jnp.arange(P, dtype=jnp.int32)
    r = jnp.where(pos < B, pos, B + ar)
    rank = jnp.zeros((B + P,), jnp.int32).at[r].add(ar, mode="drop",
                                                    unique_indices=True)
    out = jnp.take(vals, rank[:B], axis=0)
    return out.reshape(1, 1, -1)


# speedup vs baseline: 1.3917x; 1.3917x over previous
import functools

import jax
import jax.numpy as jnp
from jax import lax
from jax.experimental import pallas as pl
from jax.experimental.pallas import tpu as pltpu
from jax.experimental.pallas import tpu_sc as plsc

_W = 4096
_KW = 16
_CH = 64


def _emb_lookup_cm(idx, tab3, tail3):
    A, R, V = tab3.shape
    D = A * R
    B, = idx.shape
    info = plsc.get_sparse_core_info()
    NC, NS = info.num_cores, info.num_subcores
    a_per_c = A // NC
    n_grp = B // 16
    cap = B + 16
    cap2 = B + 256
    P = B + NS * _KW * _CH
    n_ph = a_per_c * _KW
    last_w0 = V // _W * _W
    last_len = (V - last_w0) // 128 * 128
    v_max = last_w0 + last_len
    mesh = plsc.VectorSubcoreMesh(core_axis_name="c", subcore_axis_name="s")

    @functools.partial(
        pl.kernel,
        mesh=mesh,
        out_type=(jax.ShapeDtypeStruct((D * P,), jnp.float32),
                  jax.ShapeDtypeStruct((P,), jnp.int32)),
        scratch_types=[
            pltpu.VMEM((B,), jnp.int32),
            pltpu.VMEM((cap,), jnp.int32),
            pltpu.VMEM((cap2,), jnp.int32),
            pltpu.VMEM((16,), jnp.int32),
            pltpu.VMEM((16,), jnp.int32),
            pltpu.VMEM((2, R, _W), jnp.float32),
            pltpu.VMEM((16,), jnp.int32),
            pltpu.SMEM((8,), jnp.int32),
        ] + [pltpu.VMEM((_CH,), jnp.float32) for _ in range(2 * R)] + [
            pltpu.VMEM((_CH,), jnp.int32) for _ in range(2)
        ] + [
            pltpu.SemaphoreType.DMA,
            pltpu.SemaphoreType.DMA,
        ],
        compiler_params=pltpu.CompilerParams(needs_layout_passes=False),
    )
    def emb(idx_hbm, tab_hbm, tail_hbm, vals_hbm, pos_hbm, idx_all, lst, lst2,
            off_v, cnt_v, seg_v, off2_v, smem, *rest):
        vb = [[rest[sl * R + dd] for sl in range(2)] for dd in range(R)]
        jb = [rest[2 * R], rest[2 * R + 1]]
        ssem, gsem = rest[2 * R + 2], rest[2 * R + 3]
        c = lax.axis_index("c")
        s = lax.axis_index("s")
        lanes = lax.iota(jnp.int32, 16)

        pltpu.sync_copy(idx_hbm, idx_all)

        def init(g, carry):
            lst[pl.ds(g * 16, 16)] = jnp.full((16,), B << 16, jnp.int32)
            return carry

        lax.fori_loop(0, cap // 16, init, 0)

        def scan(g, n):
            v = idx_all[pl.ds(g * 16, 16)]
            tl = v >= v_max
            m = jnp.where(tl, s == NS - 1,
                          lax.shift_right_logical(v, 16) == s)
            kf = jnp.where(tl, _KW - 1,
                           lax.shift_right_logical(v, 12) & (_KW - 1))
            loc = jnp.where(tl, v - v_max, v & (_W - 1))
            pk = lax.shift_left(lanes + g * 16, 16) | loc | (
                lax.shift_left(kf, 12))
            plsc.store_compressed(lst.at[pl.ds(n, 16)], pk, mask=m)
            return n + plsc.all_reduce_population_count(m)[0]

        n = lax.fori_loop(0, n_grp, scan, 0)
        ng = lax.shift_right_logical(n + 15, 4)

        offs = jnp.zeros((16,), jnp.int32)
        cnts = jnp.zeros((16,), jnp.int32)
        cur = 0
        for k in range(_KW):
            def split(g, m_cur, k=k):
                pk = lst[pl.ds(g * 16, 16)]
                m = (lax.shift_right_logical(pk, 12) & (_KW - 1)) == k
                plsc.store_compressed(lst2.at[pl.ds(m_cur, 16)], pk, mask=m)
                return m_cur + plsc.all_reduce_population_count(m)[0]

            nxt = lax.fori_loop(0, ng, split, cur)
            offs = jnp.where(lanes == k, cur, offs)
            cnts = jnp.where(lanes == k, nxt - cur, cnts)
            cur = nxt
        off_v[...] = offs
        cnt_v[...] = cnts

        offs2 = jnp.zeros((16,), jnp.int32)
        cur2 = 0
        for k in range(_KW):
            offs2 = jnp.where(lanes == k, cur2, offs2)
            nk2 = plsc.load_gather(cnt_v,
                                   [jnp.full((16,), k, jnp.int32)])[0]
            cur2 = cur2 + lax.shift_left(
                lax.shift_right_logical(nk2 + _CH - 1, 6), 6)
        off2_v[...] = offs2
        smem[0] = 0
        plsc.subcore_barrier()
        for t in range(NS):
            @pl.when(s < t)
            def _(t=t):
                plsc.fetch_and_add(smem.at[0], cur2, subcore_id=t)
        plsc.subcore_barrier()
        pref = smem[0]

        def stage_parts(p):
            k = p & (_KW - 1)
            kv = jnp.broadcast_to(k, (16,)).astype(jnp.int32)
            n_k = plsc.load_gather(cnt_v, [kv])[0]
            w0 = pl.multiple_of((s * _KW + k) * _W, 128)
            a = c * a_per_c + lax.shift_right_logical(p, 4)
            return n_k, w0, a

        def stage_issue(p, slot):
            n_k, w0, a = stage_parts(p)
            k = p & (_KW - 1)
            is_tail = (s == NS - 1) & (k == _KW - 1)

            @pl.when((n_k > 0) & (w0 < last_w0) & (~is_tail))
            def _():
                pltpu.async_copy(tab_hbm.at[a, :, pl.ds(w0, _W)],
                                 seg_v.at[slot], gsem)

            @pl.when((n_k > 0) & (w0 == last_w0))
            def _():
                pltpu.async_copy(
                    tab_hbm.at[a, :, pl.ds(last_w0, last_len)],
                    seg_v.at[slot, :, pl.ds(0, last_len)], gsem)

            @pl.when((n_k > 0) & is_tail)
            def _():
                pltpu.async_copy(tail_hbm.at[a],
                                 seg_v.at[slot, :, pl.ds(0, 128)], gsem)

        def stage_wait(p, slot):
            n_k, w0, a = stage_parts(p)
            k = p & (_KW - 1)
            is_tail = (s == NS - 1) & (k == _KW - 1)

            @pl.when((n_k > 0) & (w0 < last_w0) & (~is_tail))
            def _():
                pltpu.make_async_copy(tab_hbm.at[a, :, pl.ds(w0, _W)],
                                      seg_v.at[slot], gsem).wait()

            @pl.when((n_k > 0) & (w0 == last_w0))
            def _():
                pltpu.make_async_copy(
                    tab_hbm.at[a, :, pl.ds(last_w0, last_len)],
                    seg_v.at[slot, :, pl.ds(0, last_len)], gsem).wait()

            @pl.when((n_k > 0) & is_tail)
            def _():
                pltpu.make_async_copy(
                    tail_hbm.at[a],
                    seg_v.at[slot, :, pl.ds(0, 128)], gsem).wait()

        stage_issue(0, 0)

        def phase(p, pcarry):
            sg = p & 1
            stage_wait(p, sg)

            @pl.when(p + 1 < n_ph)
            def _():
                stage_issue(p + 1, 1 - sg)

            a = c * a_per_c + lax.shift_right_logical(p, 4)
            k = p & (_KW - 1)
            kvec = jnp.broadcast_to(k, (16,)).astype(jnp.int32)
            n_k = plsc.load_gather(cnt_v, [kvec])[0]
            o_k = plsc.load_gather(off_v, [kvec])[0]
            base = pl.multiple_of(
                pref + plsc.load_gather(off2_v, [kvec])[0], 64)
            nch = lax.shift_right_logical(n_k + _CH - 1, 6)
            d0 = a * R
            wpos = p < _KW
            sgv = jnp.broadcast_to(sg, (16,)).astype(jnp.int32)

            def chunk(ch, carry):
                wbase = pl.multiple_of(base + ch * _CH, 64)
                for slot in range(2):
                    @pl.when((ch & 1) == slot)
                    def _(slot=slot):
                        @pl.when(ch >= 2)
                        def _():
                            for dd in range(R):
                                pltpu.make_async_copy(
                                    vb[dd][slot],
                                    vals_hbm.at[pl.ds(wbase, _CH)],
                                    ssem,
                                ).wait()
                            @pl.when(wpos)
                            def _():
                                pltpu.make_async_copy(
                                    jb[slot],
                                    pos_hbm.at[pl.ds(wbase, _CH)],
                                    ssem).wait()

                        for g2 in range(_CH // 16):
                            e0 = ch * _CH + g2 * 16
                            pk = lst2[pl.ds(o_k + e0, 16)]
                            j = lax.shift_right_logical(pk, 16)
                            loc = pk & (_W - 1)
                            ok = ((lanes + e0) < n_k) & (j < B)
                            for dd in range(R):
                                vals = plsc.load_gather(
                                    seg_v,
                                    [sgv, jnp.full((16,), dd, jnp.int32),
                                     loc])
                                vb[dd][slot][pl.ds(g2 * 16, 16)] = vals
                            @pl.when(wpos)
                            def _(g2=g2, ok=ok, j=j):
                                jb[slot][pl.ds(g2 * 16, 16)] = jnp.where(
                                    ok, j, B)

                        for dd in range(R):
                            pltpu.async_copy(
                                vb[dd][slot],
                                vals_hbm.at[
                                    pl.ds((d0 + dd) * P + wbase, _CH)],
                                ssem,
                            )
                        @pl.when(wpos)
                        def _():
                            pltpu.async_copy(
                                jb[slot], pos_hbm.at[pl.ds(wbase, _CH)],
                                ssem)
                return carry

            lax.fori_loop(0, nch, chunk, 0)

            def drain(ch, dcarry):
                wbase = pl.multiple_of(base + ch * _CH, 64)
                for slot in range(2):
                    @pl.when((ch & 1) == slot)
                    def _(slot=slot):
                        for dd in range(R):
                            pltpu.make_async_copy(
                                vb[dd][slot],
                                vals_hbm.at[pl.ds(wbase, _CH)],
                                ssem,
                            ).wait()
                        @pl.when(wpos)
                        def _():
                            pltpu.make_async_copy(
                                jb[slot], pos_hbm.at[pl.ds(wbase, _CH)],
                                ssem).wait()
                return dcarry

            lax.fori_loop(jnp.maximum(nch - 2, 0), nch, drain, 0)
            return pcarry

        lax.fori_loop(0, n_ph, phase, 0)

    return emb(idx, tab3, tail3)


def kernel(input, table):
    idx = input.astype(jnp.int32)
    V, D = table.shape
    B, = idx.shape
    v_max = V // 4096 * 4096 + (V - V // 4096 * 4096) // 128 * 128
    tail3 = jnp.pad(table[v_max:].T,
                    ((0, 0), (0, 128 - (V - v_max)))).reshape(8, 8, 128)
    vals1, pos = _emb_lookup_cm(idx, table.T.reshape(8, 8, V), tail3)
    P = pos.shape[0]
    vals = vals1.reshape(D, P).T
    ar = jnp.arange(P, dtype=jnp.int32)
    r = jnp.where(pos < B, pos, B + ar)
    rank = jnp.zeros((B + P,), jnp.int32).at[r].add(ar, mode="drop",
                                                    unique_indices=True)
    out = jnp.take(vals, rank[:B], axis=0)
    return out.reshape(1, 1, -1)
